# baseline (speedup 1.0000x reference)
import jax
import jax.numpy as jnp
from jax.experimental import pallas as pl
from jax.experimental.pallas import tpu as pltpu

_BBLK = 16


def _ceil128(n):
    return -(-n // 128) * 128


def _band(w, s_in):
    cin, cout = w.shape[2], w.shape[3]
    wpad = s_in + 4
    sp = (s_in + 2) // 2
    wpc = sp * cout
    hh = _ceil128(wpc)
    wf = w.astype(jnp.float32)
    wv = jnp.arange(wpad)[:, None, None, None]
    tgt = (2 * jnp.arange(sp)[None, None, :, None]
           + jnp.arange(2)[None, :, None, None]
           + jnp.arange(3)[None, None, None, :])
    sel = (wv == tgt).astype(jnp.float32)
    band = jnp.einsum('wqpk,hkio->hwiqpo', sel, wf)
    band = band.reshape(3, wpad * cin, 2, wpc)
    band = jnp.pad(band, ((0, 0), (0, 0), (0, 0), (0, hh - wpc)))
    kp = _ceil128(wpad * cin)
    band = jnp.pad(band, ((0, 0), (0, kp - wpad * cin), (0, 0), (0, 0)))
    return band.reshape(3 * kp, 2 * hh).astype(jnp.bfloat16)


def _band_bias(b, s_in):
    cout = b.shape[0]
    sp = (s_in + 2) // 2
    wpc = sp * cout
    hh = _ceil128(wpc)
    bb = jnp.pad(jnp.tile(b.astype(jnp.float32), sp), (0, hh - wpc))
    return jnp.tile(bb, 2).reshape(1, 2 * hh)


def _banded_layer(xf, band_ref, bias_ref, bblk, cout, s_in, sp):
    spad = s_in + 4
    r = bblk * spad
    n2 = band_ref.shape[1]
    hh = n2 // 2
    wpc = sp * cout
    l = xf.shape[1]
    kp = _ceil128(l)
    if kp != l:
        xf = jnp.pad(xf, ((0, 0), (0, kp - l)))
    t = jnp.concatenate([xf[0:r], xf[1:r + 1], xf[2:r + 2]], axis=1)
    acc = jnp.dot(t, band_ref[...], preferred_element_type=jnp.float32)
    acc = acc + bias_ref[...]
    y = jnp.maximum(acc[:, :hh], acc[:, hh:]).astype(jnp.bfloat16)[:, :wpc]
    y = y.reshape(bblk, spad, wpc)[:, :2 * sp]
    y = y.reshape(bblk, sp, 2, wpc).max(axis=2)
    return y


def _repack_banded(p, c):
    bblk, sp, _ = p.shape
    t = jnp.pad(p, ((0, 0), (2, 2), (2 * c, 2 * c)))
    t = t.reshape(bblk * (sp + 4), (sp + 4) * c)
    return jnp.pad(t, ((0, 2), (0, 0)))


def _direct_layer(p4, w_ref, b_ref, cout, sp):
    bblk, s_in, _, cin = p4.shape
    spad = s_in + 4
    r = bblk * spad * spad
    f32 = jnp.float32
    t = jnp.pad(p4, ((0, 0), (2, 2), (2, 2), (0, 0)))
    xf = t.reshape(r, cin)
    xf = jnp.pad(xf, ((0, 2 * spad + 2), (0, 0)))
    taps = [xf[kh * spad + kw: kh * spad + kw + r]
            for kh in range(3) for kw in range(3)]
    tt = jnp.concatenate(taps, axis=1)
    acc = jnp.dot(tt, w_ref[...], preferred_element_type=f32)
    acc = (acc + b_ref[...]).astype(jnp.bfloat16)
    y = acc.reshape(bblk, spad, spad, cout)[:, :2 * sp, :2 * sp]
    y = y.reshape(bblk, sp, 2, 2 * sp, cout).max(axis=2)
    y = y.reshape(bblk, sp, sp, 2, cout).max(axis=3)
    return y


def _fused_kernel(x_ref, bd0_ref, bb0_ref, bd1_ref, bb1_ref, bd2_ref, bb2_ref,
                  w3_ref, b3_ref, w4_ref, b4_ref,
                  fw1_ref, fb1_ref, fw2_ref, fb2_ref, o_ref):
    bblk = x_ref.shape[0]
    xf = jnp.pad(x_ref[...].reshape(bblk * 36, 108), ((0, 2), (0, 0)))
    p0 = _banded_layer(xf, bd0_ref, bb0_ref, bblk, 32, 32, 17)
    p1 = _banded_layer(_repack_banded(p0, 32), bd1_ref, bb1_ref, bblk, 64, 17, 9)
    p2 = _banded_layer(_repack_banded(p1, 64), bd2_ref, bb2_ref, bblk, 128, 9, 5)
    p2w = jnp.stack([p2[:, :, w * 128:(w + 1) * 128] for w in range(5)], axis=2)
    p3 = _direct_layer(p2w, w3_ref, b3_ref, 256, 3)
    p4 = _direct_layer(p3, w4_ref, b4_ref, 512, 2)
    h = None
    for idx, (hh, ww) in enumerate(((0, 0), (0, 1), (1, 0), (1, 1))):
        d = jnp.dot(p4[:, hh, ww, :], fw1_ref[idx],
                    preferred_element_type=jnp.float32)
        h = d if h is None else h + d
    h = (h + fb1_ref[...]).astype(jnp.bfloat16)
    o_ref[...] = (jnp.dot(h, fw2_ref[...], preferred_element_type=jnp.float32)
                  + fb2_ref[...])


def kernel(x, conv0_w, conv0_b, conv1_w, conv1_b, conv2_w, conv2_b,
           conv3_w, conv3_b, conv4_w, conv4_b, fc1_w, fc1_b, fc2_w, fc2_b):
    b = x.shape[0]
    bblk = _BBLK
    xh = jnp.transpose(x, (0, 2, 3, 1)).astype(jnp.bfloat16)
    xh = jnp.pad(xh, ((0, 0), (2, 2), (2, 2), (0, 0))).reshape(b, 36, 36 * 3)

    bd0, bb0 = _band(conv0_w, 32), _band_bias(conv0_b, 32)
    bd1, bb1 = _band(conv1_w, 17), _band_bias(conv1_b, 17)
    bd2, bb2 = _band(conv2_w, 9), _band_bias(conv2_b, 9)
    w3 = conv3_w.reshape(9 * 128, 256)
    b3 = conv3_b.reshape(1, 256).astype(jnp.float32)
    w4 = conv4_w.reshape(9 * 256, 512)
    b4 = conv4_b.reshape(1, 512).astype(jnp.float32)
    fw1 = fc1_w.reshape(512, 2, 2, 1024).transpose(1, 2, 0, 3).reshape(4, 512, 1024)
    fb1 = fc1_b.reshape(1, 1024).astype(jnp.float32)
    fw2 = jnp.zeros((1024, 128), jnp.bfloat16).at[:, :10].set(fc2_w)
    fb2 = jnp.zeros((1, 128), jnp.float32).at[0, :10].set(fc2_b.astype(jnp.float32))

    vmem = pl.BlockSpec(memory_space=pltpu.VMEM)
    out = pl.pallas_call(
        _fused_kernel,
        out_shape=jax.ShapeDtypeStruct((b, 128), jnp.float32),
        grid=(b // bblk,),
        in_specs=[pl.BlockSpec((bblk, 36, 108), lambda i: (i, 0, 0))] + [vmem] * 14,
        out_specs=pl.BlockSpec((bblk, 128), lambda i: (i, 0)),
        compiler_params=pltpu.CompilerParams(
            dimension_semantics=("parallel",),
        ),
        name="fused_cnn",
    )(xh, bd0, bb0, bd1, bb1, bd2, bb2, w3, b3, w4, b4, fw1, fb1, fw2, fb2)
    return out[:, :10]


# speedup vs baseline: 1.0554x; 1.0554x over previous
import jax
import jax.numpy as jnp
from jax.experimental import pallas as pl
from jax.experimental.pallas import tpu as pltpu

_BBLK = 32


def _ceil128(n):
    return -(-n // 128) * 128


def _band(w, s_in):
    cin, cout = w.shape[2], w.shape[3]
    wpad = s_in + 4
    sp = (s_in + 2) // 2
    wpc = sp * cout
    hh = _ceil128(wpc)
    wf = w.astype(jnp.float32)
    wv = jnp.arange(wpad)[:, None, None, None]
    tgt = (2 * jnp.arange(sp)[None, None, :, None]
           + jnp.arange(2)[None, :, None, None]
           + jnp.arange(3)[None, None, None, :])
    sel = (wv == tgt).astype(jnp.float32)
    band = jnp.einsum('wqpk,hkio->hwiqpo', sel, wf)
    band = band.reshape(3, wpad * cin, 2, wpc)
    band = jnp.pad(band, ((0, 0), (0, 0), (0, 0), (0, hh - wpc)))
    kp = _ceil128(wpad * cin)
    band = jnp.pad(band, ((0, 0), (0, kp - wpad * cin), (0, 0), (0, 0)))
    return band.reshape(3 * kp, 2 * hh).astype(jnp.bfloat16)


def _band_bias(b, s_in):
    cout = b.shape[0]
    sp = (s_in + 2) // 2
    wpc = sp * cout
    hh = _ceil128(wpc)
    bb = jnp.pad(jnp.tile(b.astype(jnp.float32), sp), (0, hh - wpc))
    return jnp.tile(bb, 2).reshape(1, 2 * hh)


def _banded_matmul_pool(t, band_ref, bias_ref, bblk, cout, sp):
    hh = band_ref.shape[1] // 2
    wpc = sp * cout
    acc = jnp.dot(t, band_ref[...], preferred_element_type=jnp.float32)
    acc = acc + bias_ref[...]
    y = jnp.maximum(acc[:, :hh], acc[:, hh:]).astype(jnp.bfloat16)[:, :wpc]
    y = y.reshape(bblk, 2 * sp, wpc)
    return y.reshape(bblk, sp, 2, wpc).max(axis=2)


def _kh_pieces(p, c, s_out):
    bblk, s_in, _ = p.shape
    wpad_c = (s_in + 4) * c
    kp = _ceil128(wpad_c)
    h = 2 * s_out
    pieces = []
    for s in range(3):
        lo, hi = s - 2, s - 2 + h
        top = max(0, -lo)
        seg = p[:, max(0, lo):min(s_in, hi)]
        bot = h - top - (min(s_in, hi) - max(0, lo))
        q = jnp.pad(seg, ((0, 0), (top, bot), (2 * c, kp - wpad_c + 2 * c)))
        pieces.append(q.reshape(bblk * h, kp))
    return jnp.concatenate(pieces, axis=1)


def _direct_layer(p4, w_ref, b_ref, cout, sp):
    bblk, s_in, _, cin = p4.shape
    h = 2 * sp
    r = bblk * h * h
    t = jnp.pad(p4, ((0, 0), (2, 2), (2, 2), (0, 0)))
    pieces = [t[:, kh:kh + h, kw:kw + h, :].reshape(r, cin)
              for kh in range(3) for kw in range(3)]
    tt = jnp.concatenate(pieces, axis=1)
    acc = jnp.dot(tt, w_ref[...], preferred_element_type=jnp.float32)
    acc = (acc + b_ref[...]).astype(jnp.bfloat16)
    y = acc.reshape(bblk, h, h, cout)
    y = y.reshape(bblk, sp, 2, h, cout).max(axis=2)
    y = y.reshape(bblk, sp, sp, 2, cout).max(axis=3)
    return y


def _fused_kernel(x_ref, bd0_ref, bb0_ref, bd1_ref, bb1_ref, bd2_ref, bb2_ref,
                  w3_ref, b3_ref, w4_ref, b4_ref,
                  fw1_ref, fb1_ref, fw2_ref, fb2_ref, o_ref):
    bblk = x_ref.shape[0]
    t0 = x_ref[...].reshape(bblk * 34, 384)
    p0 = _banded_matmul_pool(t0, bd0_ref, bb0_ref, bblk, 32, 17)
    t1 = _kh_pieces(p0, 32, 9)
    p1 = _banded_matmul_pool(t1, bd1_ref, bb1_ref, bblk, 64, 9)
    t2 = _kh_pieces(p1, 64, 5)
    p2 = _banded_matmul_pool(t2, bd2_ref, bb2_ref, bblk, 128, 5)
    p2w = jnp.stack([p2[:, :, w * 128:(w + 1) * 128] for w in range(5)], axis=2)
    p3 = _direct_layer(p2w, w3_ref, b3_ref, 256, 3)
    p4 = _direct_layer(p3, w4_ref, b4_ref, 512, 2)
    h = None
    for idx, (hh, ww) in enumerate(((0, 0), (0, 1), (1, 0), (1, 1))):
        d = jnp.dot(p4[:, hh, ww, :], fw1_ref[idx],
                    preferred_element_type=jnp.float32)
        h = d if h is None else h + d
    h = (h + fb1_ref[...]).astype(jnp.bfloat16)
    o_ref[...] = (jnp.dot(h, fw2_ref[...], preferred_element_type=jnp.float32)
                  + fb2_ref[...])


def kernel(x, conv0_w, conv0_b, conv1_w, conv1_b, conv2_w, conv2_b,
           conv3_w, conv3_b, conv4_w, conv4_b, fc1_w, fc1_b, fc2_w, fc2_b):
    b = x.shape[0]
    bblk = _BBLK
    xh = jnp.transpose(x, (0, 2, 3, 1)).astype(jnp.bfloat16)
    xh = jnp.pad(xh, ((0, 0), (2, 2), (2, 2), (0, 0)))
    xh = xh.reshape(b, 36, 108)
    t0 = jnp.concatenate(
        [jnp.pad(xh[:, s:s + 34], ((0, 0), (0, 0), (0, 20))) for s in range(3)],
        axis=2)

    bd0, bb0 = _band(conv0_w, 32), _band_bias(conv0_b, 32)
    bd1, bb1 = _band(conv1_w, 17), _band_bias(conv1_b, 17)
    bd2, bb2 = _band(conv2_w, 9), _band_bias(conv2_b, 9)
    w3 = conv3_w.reshape(9 * 128, 256)
    b3 = conv3_b.reshape(1, 256).astype(jnp.float32)
    w4 = conv4_w.reshape(9 * 256, 512)
    b4 = conv4_b.reshape(1, 512).astype(jnp.float32)
    fw1 = fc1_w.reshape(512, 2, 2, 1024).transpose(1, 2, 0, 3).reshape(4, 512, 1024)
    fb1 = fc1_b.reshape(1, 1024).astype(jnp.float32)
    fw2 = jnp.zeros((1024, 128), jnp.bfloat16).at[:, :10].set(fc2_w)
    fb2 = jnp.zeros((1, 128), jnp.float32).at[0, :10].set(fc2_b.astype(jnp.float32))

    vmem = pl.BlockSpec(memory_space=pltpu.VMEM)
    out = pl.pallas_call(
        _fused_kernel,
        out_shape=jax.ShapeDtypeStruct((b, 128), jnp.float32),
        grid=(b // bblk,),
        in_specs=[pl.BlockSpec((bblk, 34, 384), lambda i: (i, 0, 0))] + [vmem] * 14,
        out_specs=pl.BlockSpec((bblk, 128), lambda i: (i, 0)),
        compiler_params=pltpu.CompilerParams(
            dimension_semantics=("parallel",),
        ),
        name="fused_cnn",
    )(t0, bd0, bb0, bd1, bb1, bd2, bb2, w3, b3, w4, b4, fw1, fb1, fw2, fb2)
    return out[:, :10]


# speedup vs baseline: 1.1573x; 1.0966x over previous
import jax
import jax.numpy as jnp
from jax.experimental import pallas as pl
from jax.experimental.pallas import tpu as pltpu

_BBLK = 32


def _ceil128(n):
    return -(-n // 128) * 128


def _band(w, s_in):
    cin, cout = w.shape[2], w.shape[3]
    wpad = s_in + 4
    sp = (s_in + 2) // 2
    wpc = sp * cout
    hh = _ceil128(wpc)
    wf = w.astype(jnp.float32)
    wv = jnp.arange(wpad)[:, None, None, None]
    tgt = (2 * jnp.arange(sp)[None, None, :, None]
           + jnp.arange(2)[None, :, None, None]
           + jnp.arange(3)[None, None, None, :])
    sel = (wv == tgt).astype(jnp.float32)
    band = jnp.einsum('wqpk,hkio->hwiqpo', sel, wf)
    band = band.reshape(3, wpad * cin, 2, wpc)
    band = jnp.pad(band, ((0, 0), (0, 0), (0, 0), (0, hh - wpc)))
    kp = _ceil128(wpad * cin)
    band = jnp.pad(band, ((0, 0), (0, kp - wpad * cin), (0, 0), (0, 0)))
    return band.reshape(3 * kp, 2 * hh).astype(jnp.bfloat16)


def _band_bias(b, s_in):
    cout = b.shape[0]
    sp = (s_in + 2) // 2
    wpc = sp * cout
    hh = _ceil128(wpc)
    bb = jnp.pad(jnp.tile(b.astype(jnp.float32), sp), (0, hh - wpc))
    return jnp.tile(bb, 2)


def _banded_matmul_pool(t, band, bias, bblk, cout, sp):
    hh = band.shape[1] // 2
    wpc = sp * cout
    acc = jnp.dot(t, band, preferred_element_type=jnp.float32)
    acc = acc + bias
    y = jnp.maximum(acc[:, :hh], acc[:, hh:])[:, :wpc]
    y = y.reshape(bblk, 2 * sp, wpc)
    y = y.reshape(bblk, sp, 2, wpc).max(axis=2)
    return y.astype(jnp.bfloat16)


def _kh_pieces(p, c, s_out):
    bblk, s_in, _ = p.shape
    wpad_c = (s_in + 4) * c
    kp = _ceil128(wpad_c)
    h = 2 * s_out
    pieces = []
    for s in range(3):
        lo, hi = s - 2, s - 2 + h
        top = max(0, -lo)
        seg = p[:, max(0, lo):min(s_in, hi)]
        bot = h - top - (min(s_in, hi) - max(0, lo))
        q = jnp.pad(seg, ((0, 0), (top, bot), (2 * c, kp - wpad_c + 2 * c)))
        pieces.append(q.reshape(bblk * h, kp))
    return jnp.concatenate(pieces, axis=1)


def _direct_layer(p4, w, bias, cout, sp):
    bblk, s_in, _, cin = p4.shape
    h = 2 * sp
    r = bblk * h * h
    t = jnp.pad(p4, ((0, 0), (2, 2), (2, 2), (0, 0)))
    pieces = [t[:, kh:kh + h, kw:kw + h, :].reshape(r, cin)
              for kh in range(3) for kw in range(3)]
    tt = jnp.concatenate(pieces, axis=1)
    acc = jnp.dot(tt, w, preferred_element_type=jnp.float32)[:, :cout]
    acc = acc + bias
    y = acc.reshape(bblk, h, h, cout)
    y = y.reshape(bblk, sp, 2, h, cout).max(axis=2)
    y = y.reshape(bblk, sp, sp, 2, cout).max(axis=3)
    return y.astype(jnp.bfloat16)


def _fused_kernel(x_ref, bd_ref, wx_ref, bs_ref, o_ref):
    bblk = x_ref.shape[0]
    t0 = x_ref[...].reshape(bblk * 34, 384)
    p0 = _banded_matmul_pool(t0, bd_ref[0:384], bs_ref[0:1], bblk, 32, 17)
    t1 = _kh_pieces(p0, 32, 9)
    p1 = _banded_matmul_pool(t1, bd_ref[384:2688], bs_ref[1:2], bblk, 64, 9)
    t2 = _kh_pieces(p1, 64, 5)
    p2 = _banded_matmul_pool(t2, bd_ref[2688:5376], bs_ref[2:3], bblk, 128, 5)
    p2w = jnp.stack([p2[:, :, w * 128:(w + 1) * 128] for w in range(5)], axis=2)
    p3 = _direct_layer(p2w, wx_ref[0:1152], bs_ref[3:4, :256], 256, 3)
    p4 = _direct_layer(p3, wx_ref[1152:3456], bs_ref[4:5, :512], 512, 2)
    ha = None
    hb = None
    for idx, (hh, ww) in enumerate(((0, 0), (0, 1), (1, 0), (1, 1))):
        xp = p4[:, hh, ww, :]
        da = jnp.dot(xp, wx_ref[3456 + 512 * idx:3456 + 512 * (idx + 1)],
                     preferred_element_type=jnp.float32)
        db = jnp.dot(xp, wx_ref[5504 + 512 * idx:5504 + 512 * (idx + 1)],
                     preferred_element_type=jnp.float32)
        ha = da if ha is None else ha + da
        hb = db if hb is None else hb + db
    hcat = jnp.concatenate([ha, hb], axis=1)
    hcat = (hcat + bs_ref[5:6, :1024]).astype(jnp.bfloat16)
    out = jnp.dot(hcat, wx_ref[7552:8576, :128],
                  preferred_element_type=jnp.float32)
    o_ref[...] = out + bs_ref[6:7, :128]


def kernel(x, conv0_w, conv0_b, conv1_w, conv1_b, conv2_w, conv2_b,
           conv3_w, conv3_b, conv4_w, conv4_b, fc1_w, fc1_b, fc2_w, fc2_b):
    b = x.shape[0]
    bblk = _BBLK
    xh = jnp.transpose(x, (0, 2, 3, 1)).astype(jnp.bfloat16)
    xh = jnp.pad(xh, ((0, 0), (2, 2), (2, 2), (0, 0)))
    xh = xh.reshape(b, 36, 108)
    t0 = jnp.concatenate(
        [jnp.pad(xh[:, s:s + 34], ((0, 0), (0, 0), (0, 20))) for s in range(3)],
        axis=2)

    bd = jnp.concatenate(
        [_band(conv0_w, 32), _band(conv1_w, 17), _band(conv2_w, 9)], axis=0)
    w3 = jnp.pad(conv3_w.reshape(9 * 128, 256), ((0, 0), (0, 256)))
    w4 = conv4_w.reshape(9 * 256, 512)
    fw1 = fc1_w.reshape(512, 2, 2, 1024).transpose(1, 2, 0, 3).reshape(2048, 1024)
    fw2 = jnp.pad(fc2_w, ((0, 0), (0, 502)))
    wx = jnp.concatenate([w3, w4, fw1[:, :512], fw1[:, 512:], fw2], axis=0)
    bs = jnp.stack([
        _band_bias(conv0_b, 32),
        _band_bias(conv1_b, 17),
        _band_bias(conv2_b, 9),
        jnp.pad(conv3_b.astype(jnp.float32), (0, 1024)),
        jnp.pad(conv4_b.astype(jnp.float32), (0, 768)),
        jnp.pad(fc1_b.astype(jnp.float32), (0, 256)),
        jnp.pad(fc2_b.astype(jnp.float32), (0, 1270)),
    ], axis=0)

    vmem = pl.BlockSpec(memory_space=pltpu.VMEM)
    out = pl.pallas_call(
        _fused_kernel,
        out_shape=jax.ShapeDtypeStruct((b, 128), jnp.float32),
        grid=(b // bblk,),
        in_specs=[pl.BlockSpec((bblk, 34, 384), lambda i: (i, 0, 0))] + [vmem] * 3,
        out_specs=pl.BlockSpec((bblk, 128), lambda i: (i, 0)),
        compiler_params=pltpu.CompilerParams(
            dimension_semantics=("parallel",),
        ),
        name="fused_cnn",
    )(t0, bd, wx, bs)
    return out[:, :10]
